# baseline (speedup 1.0000x reference)
import functools

import jax
import jax.numpy as jnp
from jax.experimental import pallas as pl
from jax.experimental.pallas import tpu as pltpu
from jax.experimental.pallas import tpu_sc as plsc

N_NODE = 10000
D = 128
E = 320000


_GW = 128


def _sc_gather2(p_mesh, p_grid, src, dst):
    ne = src.shape[0]
    mesh_sc = plsc.VectorSubcoreMesh(core_axis_name="core",
                                     subcore_axis_name="subcore")

    @functools.partial(
        pl.kernel,
        out_type=[jax.ShapeDtypeStruct((ne, D), jnp.float32),
                  jax.ShapeDtypeStruct((ne, D), jnp.float32)],
        mesh=mesh_sc,
        scratch_types=[pltpu.SemaphoreType.DMA, pltpu.SemaphoreType.DMA])
    def k(pm_hbm, pg_hbm, si_hbm, di_hbm, om_hbm, og_hbm, sem1, sem2):
        def body(si_vmem, di_vmem, om_vmem, og_vmem):
            d1 = pltpu.async_copy(pm_hbm.at[si_vmem.at[0]], om_vmem, sem1)
            d2 = pltpu.async_copy(pg_hbm.at[di_vmem.at[0]], og_vmem, sem2)
            d1.wait()
            d2.wait()

        pltpu.emit_pipeline(
            body,
            grid=(ne // _GW,),
            in_specs=[pl.BlockSpec((1, _GW), lambda i: (i, 0)),
                      pl.BlockSpec((1, _GW), lambda i: (i, 0))],
            out_specs=[pl.BlockSpec((_GW, D), lambda i: (i, 0)),
                       pl.BlockSpec((_GW, D), lambda i: (i, 0))],
            core_axis_name=("core", "subcore"),
            dimension_semantics=(pltpu.PARALLEL,),
        )(si_hbm, di_hbm, om_hbm, og_hbm)

    return k(p_mesh, p_grid, src.reshape(ne // _GW, _GW),
             dst.reshape(ne // _GW, _GW))


_NC = 2
_NS = 16
_FROWS = 624


def _sc_scatter(eout, dst, zrows):
    mesh_sc = plsc.VectorSubcoreMesh(core_axis_name="core",
                                     subcore_axis_name="subcore")
    ne = eout.shape[0]

    @functools.partial(
        pl.kernel,
        out_type=jax.ShapeDtypeStruct((_NC, N_NODE, D), jnp.float32),
        mesh=mesh_sc,
        scratch_types=[pltpu.VMEM_SHARED((N_NODE, D), jnp.float32)])
    def k(e_hbm, di_hbm, z_hbm, out_hbm, acc_sh):
        cid = jax.lax.axis_index("core")
        sid = jax.lax.axis_index("subcore")
        base = sid * _FROWS

        pltpu.sync_copy(z_hbm.at[pl.ds(0, _FROWS)],
                        acc_sh.at[pl.ds(base, _FROWS)])

        @pl.when(sid == _NS - 1)
        def _ztail():
            pltpu.sync_copy(z_hbm.at[pl.ds(0, N_NODE - _NS * _FROWS)],
                            acc_sh.at[pl.ds(_NS * _FROWS,
                                            N_NODE - _NS * _FROWS)])

        plsc.subcore_barrier()

        def body(e_vmem, di_vmem):
            pltpu.sync_copy(e_vmem, acc_sh.at[di_vmem.at[0]], add=True)

        pltpu.emit_pipeline(
            body,
            grid=(ne // _GW,),
            in_specs=[pl.BlockSpec((_GW, D), lambda i: (i, 0)),
                      pl.BlockSpec((1, _GW), lambda i: (i, 0))],
            out_specs=[],
            core_axis_name=("core", "subcore"),
            dimension_semantics=(pltpu.PARALLEL,),
        )(e_hbm, di_hbm)

        plsc.subcore_barrier()

        pltpu.sync_copy(acc_sh.at[pl.ds(base, _FROWS)],
                        out_hbm.at[cid, pl.ds(base, _FROWS)])

        @pl.when(sid == _NS - 1)
        def _ftail():
            pltpu.sync_copy(acc_sh.at[pl.ds(_NS * _FROWS,
                                            N_NODE - _NS * _FROWS)],
                            out_hbm.at[cid, pl.ds(_NS * _FROWS,
                                                  N_NODE - _NS * _FROWS)])

    return k(eout, dst.reshape(ne // _GW, _GW), zrows)


def _proj_body(m_ref, g_ref, w1b_ref, w1c_ref, pm_ref, pg_ref):
    pm_ref[...] = jnp.dot(m_ref[...], w1b_ref[...],
                          preferred_element_type=jnp.float32)
    pg_ref[...] = jnp.dot(g_ref[...], w1c_ref[...],
                          preferred_element_type=jnp.float32)


def _proj(mesh_nfeat, grid_nfeat, w1b, w1c, bn=2000):
    n = mesh_nfeat.shape[0]
    return pl.pallas_call(
        _proj_body,
        grid=(n // bn,),
        in_specs=[
            pl.BlockSpec((bn, D), lambda i: (i, 0)),
            pl.BlockSpec((bn, D), lambda i: (i, 0)),
            pl.BlockSpec((D, D), lambda i: (0, 0)),
            pl.BlockSpec((D, D), lambda i: (0, 0)),
        ],
        out_specs=[
            pl.BlockSpec((bn, D), lambda i: (i, 0)),
            pl.BlockSpec((bn, D), lambda i: (i, 0)),
        ],
        out_shape=[
            jax.ShapeDtypeStruct((n, D), jnp.float32),
            jax.ShapeDtypeStruct((n, D), jnp.float32),
        ],
    )(mesh_nfeat, grid_nfeat, w1b, w1c)


def _edge_body(ef_ref, gm_ref, gg_ref, w1a_ref, w2_ref, b1_ref, b2_ref,
               lg_ref, lb_ref, out_ref):
    pre = (jnp.dot(ef_ref[...].astype(jnp.bfloat16),
                   w1a_ref[...].astype(jnp.bfloat16),
                   preferred_element_type=jnp.float32)
           + gm_ref[...] + gg_ref[...] + b1_ref[...])
    h = pre * jax.nn.sigmoid(pre)
    o = jnp.dot(h.astype(jnp.bfloat16), w2_ref[...].astype(jnp.bfloat16),
                preferred_element_type=jnp.float32) + b2_ref[...]
    mu = jnp.mean(o, axis=-1, keepdims=True)
    var = jnp.mean((o - mu) ** 2, axis=-1, keepdims=True)
    out_ref[...] = (o - mu) * jax.lax.rsqrt(var + 1e-5) * lg_ref[...] + lb_ref[...]


def _edge_mlp(efeat, gm, gg, w1a, w2, b1, b2, lg, lb, be=3200):
    ne = efeat.shape[0]
    full = lambda i: (0, 0)
    return pl.pallas_call(
        _edge_body,
        grid=(ne // be,),
        in_specs=[
            pl.BlockSpec((be, D), lambda i: (i, 0)),
            pl.BlockSpec((be, D), lambda i: (i, 0)),
            pl.BlockSpec((be, D), lambda i: (i, 0)),
            pl.BlockSpec((D, D), full),
            pl.BlockSpec((D, D), full),
            pl.BlockSpec((1, D), full),
            pl.BlockSpec((1, D), full),
            pl.BlockSpec((1, D), full),
            pl.BlockSpec((1, D), full),
        ],
        out_specs=pl.BlockSpec((be, D), lambda i: (i, 0)),
        out_shape=jax.ShapeDtypeStruct((ne, D), jnp.float32),
    )(efeat, gm, gg, w1a, w2, b1.reshape(1, D), b2.reshape(1, D),
      lg.reshape(1, D), lb.reshape(1, D))


def _node_body(*refs):
    (g_ref, nw1a_ref, nw1b_ref, nw2_ref, nb1_ref, nb2_ref,
     lg_ref, lb_ref, out_ref) = refs[-9:]
    agg_refs = refs[:-9]
    agg = functools.reduce(
        lambda a, r: a + r[0] + r[1], agg_refs,
        jnp.zeros_like(agg_refs[0][0]))
    gf = g_ref[...]
    pre = (jnp.dot(agg, nw1a_ref[...],
                   preferred_element_type=jnp.float32)
           + jnp.dot(gf, nw1b_ref[...], preferred_element_type=jnp.float32)
           + nb1_ref[...])
    h = pre * jax.nn.sigmoid(pre)
    o = jnp.dot(h, nw2_ref[...], preferred_element_type=jnp.float32) + nb2_ref[...]
    mu = jnp.mean(o, axis=-1, keepdims=True)
    var = jnp.mean((o - mu) ** 2, axis=-1, keepdims=True)
    out_ref[...] = ((o - mu) * jax.lax.rsqrt(var + 1e-5) * lg_ref[...]
                    + lb_ref[...] + gf)


def _node_mlp(aggs, grid_nfeat, nw1a, nw1b, nw2, nb1, nb2, lg, lb, bn=2000):
    full = lambda i: (0, 0)
    return pl.pallas_call(
        _node_body,
        grid=(N_NODE // bn,),
        in_specs=[pl.BlockSpec((_NC, bn, D), lambda i: (0, i, 0))
                  for _ in aggs] + [
            pl.BlockSpec((bn, D), lambda i: (i, 0)),
            pl.BlockSpec((D, D), full),
            pl.BlockSpec((D, D), full),
            pl.BlockSpec((D, D), full),
            pl.BlockSpec((1, D), full),
            pl.BlockSpec((1, D), full),
            pl.BlockSpec((1, D), full),
            pl.BlockSpec((1, D), full),
        ],
        out_specs=pl.BlockSpec((bn, D), lambda i: (i, 0)),
        out_shape=jax.ShapeDtypeStruct((N_NODE, D), jnp.float32),
    )(*aggs, grid_nfeat, nw1a, nw1b, nw2, nb1.reshape(1, D),
      nb2.reshape(1, D), lg.reshape(1, D), lb.reshape(1, D))


def kernel(m2g_efeat, grid_nfeat, mesh_nfeat, edge_index,
           edge_w1, edge_b1, edge_w2, edge_b2, edge_ln_g, edge_ln_b,
           node_w1, node_b1, node_w2, node_b2, node_ln_g, node_ln_b):
    src = edge_index[0].astype(jnp.int32)
    dst = edge_index[1].astype(jnp.int32)
    w1a, w1b, w1c = edge_w1[:D], edge_w1[D:2 * D], edge_w1[2 * D:]
    nw1a, nw1b = node_w1[:D], node_w1[D:]

    p_mesh, p_grid = _proj(mesh_nfeat, grid_nfeat, w1b, w1c)
    zrows = jnp.zeros((_FROWS, D), jnp.float32)

    nch = 1
    ech = E // nch
    sls = [slice(c * ech, (c + 1) * ech) for c in range(nch)]
    gath = [_sc_gather2(p_mesh, p_grid, src[sl], dst[sl]) for sl in sls]
    eouts = [_edge_mlp(m2g_efeat[sl], gm, gg, w1a, edge_w2, edge_b1,
                       edge_b2, edge_ln_g, edge_ln_b)
             for sl, (gm, gg) in zip(sls, gath)]
    aggs = [_sc_scatter(eo, dst[sl], zrows)
            for sl, eo in zip(sls, eouts)]

    return _node_mlp(aggs, grid_nfeat, nw1a, nw1b, node_w2, node_b1, node_b2,
                     node_ln_g, node_ln_b)


# speedup vs baseline: 1.0410x; 1.0410x over previous
import functools

import jax
import jax.numpy as jnp
from jax.experimental import pallas as pl
from jax.experimental.pallas import tpu as pltpu
from jax.experimental.pallas import tpu_sc as plsc

N_NODE = 10000
D = 128
E = 320000


_GW = 128


def _sc_gather2(p_mesh, p_grid, src, dst):
    ne = src.shape[0]
    mesh_sc = plsc.VectorSubcoreMesh(core_axis_name="core",
                                     subcore_axis_name="subcore")

    @functools.partial(
        pl.kernel,
        out_type=[jax.ShapeDtypeStruct((ne, D), jnp.float32),
                  jax.ShapeDtypeStruct((ne, D), jnp.float32)],
        mesh=mesh_sc,
        scratch_types=[pltpu.SemaphoreType.DMA, pltpu.SemaphoreType.DMA])
    def k(pm_hbm, pg_hbm, si_hbm, di_hbm, om_hbm, og_hbm, sem1, sem2):
        def body(si_vmem, di_vmem, om_vmem, og_vmem):
            d1 = pltpu.async_copy(pm_hbm.at[si_vmem.at[0]], om_vmem, sem1)
            d2 = pltpu.async_copy(pg_hbm.at[di_vmem.at[0]], og_vmem, sem2)
            d1.wait()
            d2.wait()

        pltpu.emit_pipeline(
            body,
            grid=(ne // _GW,),
            in_specs=[pl.BlockSpec((1, _GW), lambda i: (i, 0)),
                      pl.BlockSpec((1, _GW), lambda i: (i, 0))],
            out_specs=[pl.BlockSpec((_GW, D), lambda i: (i, 0)),
                       pl.BlockSpec((_GW, D), lambda i: (i, 0))],
            core_axis_name=("core", "subcore"),
            dimension_semantics=(pltpu.PARALLEL,),
        )(si_hbm, di_hbm, om_hbm, og_hbm)

    return k(p_mesh, p_grid, src.reshape(ne // _GW, _GW),
             dst.reshape(ne // _GW, _GW))


_NC = 2
_NS = 16
_FROWS = 624


def _sc_scatter(eout, dst, zrows):
    mesh_sc = plsc.VectorSubcoreMesh(core_axis_name="core",
                                     subcore_axis_name="subcore")
    ne = eout.shape[0]

    @functools.partial(
        pl.kernel,
        out_type=jax.ShapeDtypeStruct((_NC, N_NODE, D), jnp.float32),
        mesh=mesh_sc,
        scratch_types=[pltpu.VMEM_SHARED((N_NODE, D), jnp.float32)])
    def k(e_hbm, di_hbm, z_hbm, out_hbm, acc_sh):
        cid = jax.lax.axis_index("core")
        sid = jax.lax.axis_index("subcore")
        base = sid * _FROWS

        pltpu.sync_copy(z_hbm.at[pl.ds(0, _FROWS)],
                        acc_sh.at[pl.ds(base, _FROWS)])

        @pl.when(sid == _NS - 1)
        def _ztail():
            pltpu.sync_copy(z_hbm.at[pl.ds(0, N_NODE - _NS * _FROWS)],
                            acc_sh.at[pl.ds(_NS * _FROWS,
                                            N_NODE - _NS * _FROWS)])

        plsc.subcore_barrier()

        def body(e_vmem, di_vmem):
            pltpu.sync_copy(e_vmem, acc_sh.at[di_vmem.at[0]], add=True)

        pltpu.emit_pipeline(
            body,
            grid=(ne // _GW,),
            in_specs=[pl.BlockSpec((_GW, D), lambda i: (i, 0)),
                      pl.BlockSpec((1, _GW), lambda i: (i, 0))],
            out_specs=[],
            core_axis_name=("core", "subcore"),
            dimension_semantics=(pltpu.PARALLEL,),
        )(e_hbm, di_hbm)

        plsc.subcore_barrier()

        pltpu.sync_copy(acc_sh.at[pl.ds(base, _FROWS)],
                        out_hbm.at[cid, pl.ds(base, _FROWS)])

        @pl.when(sid == _NS - 1)
        def _ftail():
            pltpu.sync_copy(acc_sh.at[pl.ds(_NS * _FROWS,
                                            N_NODE - _NS * _FROWS)],
                            out_hbm.at[cid, pl.ds(_NS * _FROWS,
                                                  N_NODE - _NS * _FROWS)])

    return k(eout, dst.reshape(ne // _GW, _GW), zrows)


def _proj_body(m_ref, g_ref, w1b_ref, w1c_ref, pm_ref, pg_ref):
    pm_ref[...] = jnp.dot(m_ref[...], w1b_ref[...],
                          preferred_element_type=jnp.float32)
    pg_ref[...] = jnp.dot(g_ref[...], w1c_ref[...],
                          preferred_element_type=jnp.float32)


def _proj(mesh_nfeat, grid_nfeat, w1b, w1c, bn=2000):
    n = mesh_nfeat.shape[0]
    return pl.pallas_call(
        _proj_body,
        grid=(n // bn,),
        in_specs=[
            pl.BlockSpec((bn, D), lambda i: (i, 0)),
            pl.BlockSpec((bn, D), lambda i: (i, 0)),
            pl.BlockSpec((D, D), lambda i: (0, 0)),
            pl.BlockSpec((D, D), lambda i: (0, 0)),
        ],
        out_specs=[
            pl.BlockSpec((bn, D), lambda i: (i, 0)),
            pl.BlockSpec((bn, D), lambda i: (i, 0)),
        ],
        out_shape=[
            jax.ShapeDtypeStruct((n, D), jnp.float32),
            jax.ShapeDtypeStruct((n, D), jnp.float32),
        ],
    )(mesh_nfeat, grid_nfeat, w1b, w1c)


def _edge_body(ef_ref, gm_ref, gg_ref, w1a_ref, w2_ref, b1_ref, b2_ref,
               lg_ref, lb_ref, out_ref):
    pre = (jnp.dot(ef_ref[...], w1a_ref[...],
                   preferred_element_type=jnp.float32)
           + gm_ref[...] + gg_ref[...] + b1_ref[...])
    h = pre * jax.nn.sigmoid(pre)
    o = jnp.dot(h, w2_ref[...], preferred_element_type=jnp.float32) + b2_ref[...]
    mu = jnp.mean(o, axis=-1, keepdims=True)
    var = jnp.mean((o - mu) ** 2, axis=-1, keepdims=True)
    out_ref[...] = (o - mu) * jax.lax.rsqrt(var + 1e-5) * lg_ref[...] + lb_ref[...]


def _edge_mlp(efeat, gm, gg, w1a, w2, b1, b2, lg, lb, be=6400):
    ne = efeat.shape[0]
    full = lambda i: (0, 0)
    return pl.pallas_call(
        _edge_body,
        grid=(ne // be,),
        in_specs=[
            pl.BlockSpec((be, D), lambda i: (i, 0)),
            pl.BlockSpec((be, D), lambda i: (i, 0)),
            pl.BlockSpec((be, D), lambda i: (i, 0)),
            pl.BlockSpec((D, D), full),
            pl.BlockSpec((D, D), full),
            pl.BlockSpec((1, D), full),
            pl.BlockSpec((1, D), full),
            pl.BlockSpec((1, D), full),
            pl.BlockSpec((1, D), full),
        ],
        out_specs=pl.BlockSpec((be, D), lambda i: (i, 0)),
        out_shape=jax.ShapeDtypeStruct((ne, D), jnp.float32),
    )(efeat, gm, gg, w1a, w2, b1.reshape(1, D), b2.reshape(1, D),
      lg.reshape(1, D), lb.reshape(1, D))


def _node_body(*refs):
    (g_ref, nw1a_ref, nw1b_ref, nw2_ref, nb1_ref, nb2_ref,
     lg_ref, lb_ref, out_ref) = refs[-9:]
    agg_refs = refs[:-9]
    agg = functools.reduce(
        lambda a, r: a + r[0] + r[1], agg_refs,
        jnp.zeros_like(agg_refs[0][0]))
    gf = g_ref[...]
    pre = (jnp.dot(agg, nw1a_ref[...],
                   preferred_element_type=jnp.float32)
           + jnp.dot(gf, nw1b_ref[...], preferred_element_type=jnp.float32)
           + nb1_ref[...])
    h = pre * jax.nn.sigmoid(pre)
    o = jnp.dot(h, nw2_ref[...], preferred_element_type=jnp.float32) + nb2_ref[...]
    mu = jnp.mean(o, axis=-1, keepdims=True)
    var = jnp.mean((o - mu) ** 2, axis=-1, keepdims=True)
    out_ref[...] = ((o - mu) * jax.lax.rsqrt(var + 1e-5) * lg_ref[...]
                    + lb_ref[...] + gf)


def _node_mlp(aggs, grid_nfeat, nw1a, nw1b, nw2, nb1, nb2, lg, lb, bn=2000):
    full = lambda i: (0, 0)
    return pl.pallas_call(
        _node_body,
        grid=(N_NODE // bn,),
        in_specs=[pl.BlockSpec((_NC, bn, D), lambda i: (0, i, 0))
                  for _ in aggs] + [
            pl.BlockSpec((bn, D), lambda i: (i, 0)),
            pl.BlockSpec((D, D), full),
            pl.BlockSpec((D, D), full),
            pl.BlockSpec((D, D), full),
            pl.BlockSpec((1, D), full),
            pl.BlockSpec((1, D), full),
            pl.BlockSpec((1, D), full),
            pl.BlockSpec((1, D), full),
        ],
        out_specs=pl.BlockSpec((bn, D), lambda i: (i, 0)),
        out_shape=jax.ShapeDtypeStruct((N_NODE, D), jnp.float32),
    )(*aggs, grid_nfeat, nw1a, nw1b, nw2, nb1.reshape(1, D),
      nb2.reshape(1, D), lg.reshape(1, D), lb.reshape(1, D))


def kernel(m2g_efeat, grid_nfeat, mesh_nfeat, edge_index,
           edge_w1, edge_b1, edge_w2, edge_b2, edge_ln_g, edge_ln_b,
           node_w1, node_b1, node_w2, node_b2, node_ln_g, node_ln_b):
    src = edge_index[0].astype(jnp.int32)
    dst = edge_index[1].astype(jnp.int32)
    w1a, w1b, w1c = edge_w1[:D], edge_w1[D:2 * D], edge_w1[2 * D:]
    nw1a, nw1b = node_w1[:D], node_w1[D:]

    p_mesh, p_grid = _proj(mesh_nfeat, grid_nfeat, w1b, w1c)
    zrows = jnp.zeros((_FROWS, D), jnp.float32)

    nch = 1
    ech = E // nch
    sls = [slice(c * ech, (c + 1) * ech) for c in range(nch)]
    gath = [_sc_gather2(p_mesh, p_grid, src[sl], dst[sl]) for sl in sls]
    eouts = [_edge_mlp(m2g_efeat[sl], gm, gg, w1a, edge_w2, edge_b1,
                       edge_b2, edge_ln_g, edge_ln_b)
             for sl, (gm, gg) in zip(sls, gath)]
    aggs = [_sc_scatter(eo, dst[sl], zrows)
            for sl, eo in zip(sls, eouts)]

    return _node_mlp(aggs, grid_nfeat, nw1a, nw1b, node_w2, node_b1, node_b2,
                     node_ln_g, node_ln_b)
